# baseline (speedup 1.0000x reference)
import functools

import jax
import jax.numpy as jnp
from jax import lax
from jax.experimental import pallas as pl
from jax.experimental.pallas import tpu as pltpu
from jax.experimental.pallas import tpu_sc as plsc

D_MODEL = 64
SCALE = 8.0

_NC = 2
_NS = 16
_NW = _NC * _NS

_C = 128
_GROUPS = _C * D_MODEL // 16


def _emb_body(x_hbm, lut_hbm, out_hbm,
              idx0, idx1, rows0, rows1,
              si0, si1, sg0, sg1, so0, so1,
              *, bpw, nchunk):
  idx = (idx0, idx1)
  rows = (rows0, rows1)
  sidx = (si0, si1)
  sg = (sg0, sg1)
  so = (so0, so1)

  wid = lax.axis_index("s") * _NC + lax.axis_index("c")
  base = wid * bpw

  def idx_copy(c, s):
    return pltpu.make_async_copy(
        x_hbm.at[pl.ds(base + c * _C, _C)], idx[s], sidx[s])

  def gather_copy(s):
    return pltpu.make_async_copy(lut_hbm.at[idx[s]], rows[s], sg[s])

  def out_copy(c, s):
    return pltpu.make_async_copy(
        rows[s], out_hbm.at[pl.ds(base + c * _C, _C)], so[s])

  idx_copy(0, 0).start()
  idx_copy(1, 1).start()
  idx_copy(0, 0).wait()
  gather_copy(0).start()

  @pl.loop(0, nchunk, step=2)
  def _chunk_pair(c):
    for b in range(2):
      cc = c + b
      s, t = b, 1 - b

      gather_copy(s).wait()

      @pl.when(cc + 2 < nchunk)
      def _():
        idx_copy(cc + 2, s).start()

      @pl.when(cc + 1 < nchunk)
      def _():
        @pl.when(cc >= 1)
        def _():
          out_copy(cc - 1, t).wait()
        idx_copy(cc + 1, t).wait()
        gather_copy(t).start()

      @pl.loop(0, _C)
      def _scale_row(r):
        for j in range(D_MODEL // 16):
          sl = pl.ds(j * 16, 16)
          rows[s][r, sl] = rows[s][r, sl] * SCALE

      out_copy(cc, s).start()

  out_copy(nchunk - 2, 0).wait()
  out_copy(nchunk - 1, 1).wait()


@jax.jit
def kernel(x, lut):
  orig_shape = x.shape
  xf = x.reshape(-1).astype(jnp.int32)
  n = xf.shape[0]
  assert n % (_NW * _C) == 0
  bpw = n // _NW
  nchunk = bpw // _C

  mesh = plsc.VectorSubcoreMesh(core_axis_name="c", subcore_axis_name="s")
  body = functools.partial(_emb_body, bpw=bpw, nchunk=nchunk)
  out = pl.kernel(
      body,
      out_type=jax.ShapeDtypeStruct((n, D_MODEL), jnp.float32),
      mesh=mesh,
      compiler_params=pltpu.CompilerParams(use_tc_tiling_on_sc=False),
      scratch_types=[
          pltpu.VMEM((_C,), jnp.int32),
          pltpu.VMEM((_C,), jnp.int32),
          pltpu.VMEM((_C, D_MODEL), jnp.float32),
          pltpu.VMEM((_C, D_MODEL), jnp.float32),
          pltpu.SemaphoreType.DMA,
          pltpu.SemaphoreType.DMA,
          pltpu.SemaphoreType.DMA,
          pltpu.SemaphoreType.DMA,
          pltpu.SemaphoreType.DMA,
          pltpu.SemaphoreType.DMA,
      ],
  )(xf, lut)
  return out.reshape(*orig_shape, D_MODEL)
